# baseline (speedup 1.0000x reference)
import functools

import jax
import jax.numpy as jnp
from jax import lax
from jax.experimental import pallas as pl
from jax.experimental.pallas import tpu as pltpu
from jax.experimental.pallas import tpu_sc as plsc

_NUM_EMBEDDINGS = 1000001
_EMBED_DIM = 32
_BATCH = 16384
_CHUNK = 128


@functools.cache
def _build_gather():
    info = plsc.get_sparse_core_info()
    num_cores, num_subcores = info.num_cores, info.num_subcores
    num_workers = num_cores * num_subcores
    b_per_w = _BATCH // num_workers
    n_chunks = b_per_w // _CHUNK
    mesh = plsc.VectorSubcoreMesh(core_axis_name="c", subcore_axis_name="s")

    @functools.partial(
        pl.kernel,
        out_type=jax.ShapeDtypeStruct((_BATCH, _EMBED_DIM), jnp.float32),
        mesh=mesh,
        scratch_types=[
            pltpu.VMEM((b_per_w,), jnp.int32),
            pltpu.VMEM((b_per_w, _EMBED_DIM), jnp.float32),
            pltpu.SemaphoreType.DMA,
        ],
        compiler_params=pltpu.CompilerParams(use_tc_tiling_on_sc=False),
    )
    def gather_kernel(ids_hbm, table_hbm, out_hbm, idx_v, rows_v, sem):
        wid = lax.axis_index("s") * num_cores + lax.axis_index("c")
        base = wid * b_per_w
        pltpu.sync_copy(ids_hbm.at[pl.ds(base, b_per_w)], idx_v)
        copies = []
        for j in range(n_chunks):
            sl = pl.ds(j * _CHUNK, _CHUNK)
            copies.append(
                pltpu.async_copy(table_hbm.at[idx_v.at[sl]], rows_v.at[sl], sem)
            )
        for c in copies:
            c.wait()
        pltpu.sync_copy(rows_v, out_hbm.at[pl.ds(base, b_per_w)])

    return gather_kernel


@jax.jit
def kernel(class_ids, table):
    return _build_gather()(class_ids.astype(jnp.int32), table)


# speedup vs baseline: 3.7763x; 3.7763x over previous
import functools

import jax
import jax.numpy as jnp
from jax import lax
from jax.experimental import pallas as pl
from jax.experimental.pallas import tpu as pltpu
from jax.experimental.pallas import tpu_sc as plsc

_VOCAB = 1000001
_DIM = 32
_BATCH = 16384
_K = 8


@functools.cache
def _build_gather():
    info = plsc.get_sparse_core_info()
    num_cores, num_subcores = info.num_cores, info.num_subcores
    num_workers = num_cores * num_subcores
    b_per_w = _BATCH // num_workers
    n_steps = b_per_w // 16
    mesh = plsc.VectorSubcoreMesh(core_axis_name="c", subcore_axis_name="s")

    @functools.partial(
        pl.kernel,
        out_type=jax.ShapeDtypeStruct((_DIM, _BATCH), jnp.float32),
        mesh=mesh,
        scratch_types=[
            pltpu.VMEM((b_per_w,), jnp.int32),
            pltpu.VMEM((2, _K, _DIM, 128), jnp.float32),
            pltpu.VMEM((_DIM, b_per_w), jnp.float32),
            pltpu.SemaphoreType.DMA,
            pltpu.SemaphoreType.DMA,
        ],
        compiler_params=pltpu.CompilerParams(needs_layout_passes=False),
    )
    def gather_kernel(ids_hbm, tab_hbm, out_hbm, ids_v, blocks, stage,
                      sem0, sem1):
        wid = lax.axis_index("s") * num_cores + lax.axis_index("c")
        base = wid * b_per_w
        pltpu.sync_copy(ids_hbm.at[pl.ds(base, b_per_w)], ids_v)

        iota = lax.iota(jnp.int32, 16)
        sems = (sem0, sem1)

        def fire(chunk, half, parity):
            for k in range(_K):
                r = chunk[half * _K + k]
                o = pl.multiple_of(r & ~jnp.int32(127), 128)
                pltpu.async_copy(
                    tab_hbm.at[:, pl.ds(o, 128)],
                    blocks.at[parity, k], sems[parity])

        def drain(parity):
            for k in range(_K):
                pltpu.make_async_copy(
                    tab_hbm.at[:, pl.ds(0, 128)],
                    blocks.at[parity, k], sems[parity]).wait()

        def extract(chunk, half, col0, parity):
            for k in range(_K):
                r = chunk[half * _K + k]
                dl = r & jnp.int32(127)
                l_idx = jnp.full((16,), dl, jnp.int32)
                col = jnp.full((16,), col0 + half * _K + k, jnp.int32)
                blk = blocks.at[parity, k]
                v0 = plsc.load_gather(blk, [iota, l_idx])
                v1 = plsc.load_gather(blk, [iota + 16, l_idx])
                plsc.store_scatter(stage, [iota, col], v0)
                plsc.store_scatter(stage, [iota + 16, col], v1)

        chunk0 = ids_v[pl.ds(0, 16)]
        fire(chunk0, 0, 0)
        fire(chunk0, 1, 1)

        def body(g, carry):
            cur = ids_v[pl.ds(g * 16, 16)]
            nxt = ids_v[pl.ds((g + 1) * 16, 16)]
            col0 = g * 16
            drain(0)
            extract(cur, 0, col0, 0)
            fire(nxt, 0, 0)
            drain(1)
            extract(cur, 1, col0, 1)
            fire(nxt, 1, 1)
            return carry

        lax.fori_loop(0, n_steps - 1, body, 0)
        last = ids_v[pl.ds((n_steps - 1) * 16, 16)]
        col0 = (n_steps - 1) * 16
        drain(0)
        extract(last, 0, col0, 0)
        drain(1)
        extract(last, 1, col0, 1)

        pltpu.sync_copy(stage, out_hbm.at[:, pl.ds(base, b_per_w)])

    return gather_kernel


@jax.jit
def kernel(class_ids, table):
    out_t = _build_gather()(class_ids.astype(jnp.int32), table.T)
    return out_t.T


# speedup vs baseline: 4.0593x; 1.0750x over previous
import functools

import jax
import jax.numpy as jnp
from jax import lax
from jax.experimental import pallas as pl
from jax.experimental.pallas import tpu as pltpu
from jax.experimental.pallas import tpu_sc as plsc

_VOCAB = 1000001
_DIM = 32
_BATCH = 16384
_K = 8


@functools.cache
def _build_gather():
    info = plsc.get_sparse_core_info()
    num_cores, num_subcores = info.num_cores, info.num_subcores
    num_workers = num_cores * num_subcores
    b_per_w = _BATCH // num_workers
    n_steps = b_per_w // 16
    mesh = plsc.VectorSubcoreMesh(core_axis_name="c", subcore_axis_name="s")

    @functools.partial(
        pl.kernel,
        out_type=jax.ShapeDtypeStruct((_DIM, _BATCH), jnp.float32),
        mesh=mesh,
        scratch_types=[
            pltpu.VMEM((b_per_w,), jnp.int32),
            pltpu.VMEM((3, _K, _DIM, 128), jnp.float32),
            pltpu.VMEM((_DIM, b_per_w), jnp.float32),
            pltpu.SemaphoreType.DMA,
            pltpu.SemaphoreType.DMA,
            pltpu.SemaphoreType.DMA,
        ],
        compiler_params=pltpu.CompilerParams(needs_layout_passes=False),
    )
    def gather_kernel(ids_hbm, tab_hbm, out_hbm, ids_v, blocks, stage,
                      sem0, sem1, sem2):
        wid = lax.axis_index("s") * num_cores + lax.axis_index("c")
        base = wid * b_per_w
        pltpu.sync_copy(ids_hbm.at[pl.ds(base, b_per_w)], ids_v)

        iota = lax.iota(jnp.int32, 16)
        sems = (sem0, sem1, sem2)

        def chunk_of(t):
            return ids_v[pl.ds((t // 2) * 16, 16)]

        def fire(t, half, parity):
            chunk = chunk_of(t)
            for k in range(_K):
                r = chunk[half * _K + k]
                o = pl.multiple_of(r & ~jnp.int32(127), 128)
                pltpu.async_copy(
                    tab_hbm.at[:, pl.ds(o, 128)],
                    blocks.at[parity, k], sems[parity])

        def drain(parity):
            for k in range(_K):
                pltpu.make_async_copy(
                    tab_hbm.at[:, pl.ds(0, 128)],
                    blocks.at[parity, k], sems[parity]).wait()

        def extract(t, half, parity):
            chunk = chunk_of(t)
            for k in range(_K):
                r = chunk[half * _K + k]
                dl = r & jnp.int32(127)
                l_idx = jnp.full((16,), dl, jnp.int32)
                col = jnp.full((16,), t * _K + k, jnp.int32)
                blk = blocks.at[parity, k]
                v0 = plsc.load_gather(blk, [iota, l_idx])
                v1 = plsc.load_gather(blk, [iota + 16, l_idx])
                plsc.store_scatter(stage, [iota, col], v0)
                plsc.store_scatter(stage, [iota + 16, col], v1)

        n_batches = b_per_w // _K
        fire(0, 0, 0)
        fire(1, 1, 1)
        fire(2, 0, 2)

        def body(g, carry):
            t0 = g * 6
            for j in range(6):
                t = t0 + j
                parity = j % 3
                drain(parity)
                extract(t, j % 2, parity)
                fire(t + 3, (j + 3) % 2, parity)
            return carry

        lax.fori_loop(0, n_batches // 6, body, 0)
        drain(0)
        extract(60, 0, 0)
        fire(63, 1, 0)
        drain(1)
        extract(61, 1, 1)
        drain(2)
        extract(62, 0, 2)
        drain(0)
        extract(63, 1, 0)

        pltpu.sync_copy(stage, out_hbm.at[:, pl.ds(base, b_per_w)])

    return gather_kernel


@jax.jit
def kernel(class_ids, table):
    out_t = _build_gather()(class_ids.astype(jnp.int32), table.T)
    return out_t.T


# speedup vs baseline: 4.0713x; 1.0030x over previous
import functools

import jax
import jax.numpy as jnp
from jax import lax
from jax.experimental import pallas as pl
from jax.experimental.pallas import tpu as pltpu
from jax.experimental.pallas import tpu_sc as plsc

_VOCAB = 1000001
_DIM = 32
_BATCH = 16384
_K = 8


@functools.cache
def _build_gather():
    info = plsc.get_sparse_core_info()
    num_cores, num_subcores = info.num_cores, info.num_subcores
    num_workers = num_cores * num_subcores
    b_per_w = _BATCH // num_workers
    n_steps = b_per_w // 16
    mesh = plsc.VectorSubcoreMesh(core_axis_name="c", subcore_axis_name="s")

    @functools.partial(
        pl.kernel,
        out_type=jax.ShapeDtypeStruct((_DIM, _BATCH), jnp.float32),
        mesh=mesh,
        scratch_types=[
            pltpu.VMEM((b_per_w,), jnp.int32),
            pltpu.VMEM((3, _K, _DIM, 128), jnp.float32),
            pltpu.VMEM((_DIM, b_per_w), jnp.float32),
            pltpu.SemaphoreType.DMA,
            pltpu.SemaphoreType.DMA,
            pltpu.SemaphoreType.DMA,
        ],
        compiler_params=pltpu.CompilerParams(needs_layout_passes=False),
    )
    def gather_kernel(ids_hbm, tab_hbm, out_hbm, ids_v, blocks, stage,
                      sem0, sem1, sem2):
        wid = lax.axis_index("s") * num_cores + lax.axis_index("c")
        base = wid * b_per_w
        pltpu.sync_copy(ids_hbm.at[pl.ds(base, b_per_w)], ids_v)

        iota = lax.iota(jnp.int32, 16)
        sems = (sem0, sem1, sem2)

        def chunk_of(t):
            return ids_v[pl.ds((t // 2) * 16, 16)]

        def fire(t, half, parity):
            chunk = chunk_of(t)
            for k in range(_K):
                r = chunk[half * _K + k]
                o = pl.multiple_of(r & ~jnp.int32(127), 128)
                pltpu.async_copy(
                    tab_hbm.at[:, pl.ds(o, 128)],
                    blocks.at[parity, k], sems[parity])

        def drain(parity):
            for k in range(_K):
                pltpu.make_async_copy(
                    tab_hbm.at[:, pl.ds(0, 128)],
                    blocks.at[parity, k], sems[parity]).wait()

        def extract(t, half, parity):
            chunk = chunk_of(t)
            for k in range(_K):
                r = chunk[half * _K + k]
                dl = r & jnp.int32(127)
                l_idx = jnp.full((16,), dl, jnp.int32)
                col = jnp.full((16,), t * _K + k, jnp.int32)
                blk = blocks.at[parity, k]
                v0 = plsc.load_gather(blk, [iota, l_idx])
                v1 = plsc.load_gather(blk, [iota + 16, l_idx])
                plsc.store_scatter(stage, [iota, col], v0)
                plsc.store_scatter(stage, [iota + 16, col], v1)

        n_batches = b_per_w // _K
        assert n_batches == 64
        fire(0, 0, 0)
        fire(1, 1, 1)
        fire(2, 0, 2)

        def body(g, carry):
            t0 = g * 6
            for j in range(6):
                t = t0 + j
                parity = j % 3
                drain(parity)
                extract(t, j % 2, parity)
                fire(t + 3, (j + 3) % 2, parity)
            return carry

        lax.fori_loop(0, n_batches // 6, body, 0)
        drain(0)
        extract(60, 0, 0)
        fire(63, 1, 0)
        drain(1)
        extract(61, 1, 1)
        drain(2)
        extract(62, 0, 2)
        drain(0)
        extract(63, 1, 0)

        pltpu.sync_copy(stage, out_hbm.at[:, pl.ds(base, b_per_w)])

    return gather_kernel


@jax.jit
def kernel(class_ids, table):
    out_t = _build_gather()(class_ids.astype(jnp.int32), table.T)
    return out_t.T
